# baseline (speedup 1.0000x reference)
import functools

import jax
import jax.numpy as jnp
from jax import lax
from jax.experimental import pallas as pl
from jax.experimental.pallas import tpu as pltpu
from jax.experimental.pallas import tpu_sc as plsc

B = 1024
N = 64
D = 128
TOTAL = B * N
NC, NS, L = 2, 16, 16
NW = NC * NS
PER_W = TOTAL // NW
C = 128
NCH = PER_W // C


def _compute_chunk(h_rows, r_rows, t_rows, part_v, out_v, c, lane_iota):
    def group_body(g, _):
        def row_body(rr, _):
            i = g * L + rr
            acc = (h_rows[i, pl.ds(0, L)]
                   * r_rows[i, pl.ds(0, L)]
                   * t_rows[i, pl.ds(0, L)])
            for j in range(1, D // L):
                acc = acc + (h_rows[i, pl.ds(j * L, L)]
                             * r_rows[i, pl.ds(j * L, L)]
                             * t_rows[i, pl.ds(j * L, L)])
            plsc.store_scatter(part_v, [lane_iota * L + rr], acc)
            return 0

        lax.fori_loop(0, L, row_body, 0, unroll=4)

        tot = part_v[pl.ds(0, L)]
        for k in range(1, L):
            tot = tot + part_v[pl.ds(k * L, L)]
        out_v[pl.ds(c * C + g * L, L)] = tot
        return 0

    lax.fori_loop(0, C // L, group_body, 0)


def _sc_body(data_hbm, e_hbm, rel_hbm, out_hbm,
             raw_v, hi_all, ri_all, ti_all, bufs, part_v, out_v,
             sem_idx, sems):
    wid = lax.axis_index("s") * NC + lax.axis_index("c")
    base = wid * PER_W
    lane_iota = lax.iota(jnp.int32, L)

    pltpu.async_copy(data_hbm.at[pl.ds(base * 3, PER_W * 3)], raw_v,
                     sem_idx).wait()

    stride3 = lane_iota * 3
    dsts = (hi_all, ri_all, ti_all)

    def deint_body(k, _):
        sl = pl.ds(k * L, L)
        for f in range(3):
            dsts[f][sl] = plsc.load_gather(raw_v, [stride3 + (k * (3 * L) + f)])
        return 0

    lax.fori_loop(0, PER_W // L, deint_body, 0, unroll=4)

    def fire(c, b):
        sl = pl.ds(c * C, C)
        h_rows, r_rows, t_rows = bufs[b]
        return [
            pltpu.async_copy(e_hbm.at[hi_all.at[sl]], h_rows, sems[b]),
            pltpu.async_copy(rel_hbm.at[ri_all.at[sl]], r_rows, sems[b]),
            pltpu.async_copy(e_hbm.at[ti_all.at[sl]], t_rows, sems[b]),
        ]

    pending = {0: fire(0, 0)}
    for c in range(NCH):
        b = c % 2
        if c + 1 < NCH:
            pending[c + 1] = fire(c + 1, 1 - b)
        for cp in pending.pop(c):
            cp.wait()
        h_rows, r_rows, t_rows = bufs[b]
        _compute_chunk(h_rows, r_rows, t_rows, part_v, out_v, c, lane_iota)

    pltpu.sync_copy(out_v, out_hbm.at[pl.ds(base, PER_W)])


@functools.partial(
    pl.kernel,
    out_type=jax.ShapeDtypeStruct((TOTAL,), jnp.float32),
    mesh=plsc.VectorSubcoreMesh(core_axis_name="c", subcore_axis_name="s"),
    compiler_params=pltpu.CompilerParams(needs_layout_passes=False),
    scratch_types=[
        pltpu.VMEM((PER_W * 3,), jnp.int32),
        pltpu.VMEM((PER_W,), jnp.int32),
        pltpu.VMEM((PER_W,), jnp.int32),
        pltpu.VMEM((PER_W,), jnp.int32),
        pltpu.VMEM((C, D), jnp.float32),
        pltpu.VMEM((C, D), jnp.float32),
        pltpu.VMEM((C, D), jnp.float32),
        pltpu.VMEM((C, D), jnp.float32),
        pltpu.VMEM((C, D), jnp.float32),
        pltpu.VMEM((C, D), jnp.float32),
        pltpu.VMEM((L * L,), jnp.float32),
        pltpu.VMEM((PER_W,), jnp.float32),
        pltpu.SemaphoreType.DMA,
        pltpu.SemaphoreType.DMA,
        pltpu.SemaphoreType.DMA,
    ],
)
def _distmult_sc(data_hbm, e_hbm, rel_hbm, out_hbm,
                 raw_v, hi_all, ri_all, ti_all,
                 h0, r0, t0, h1, r1, t1, part_v, out_v,
                 sem_idx, sem_a, sem_b):
    _sc_body(data_hbm, e_hbm, rel_hbm, out_hbm,
             raw_v, hi_all, ri_all, ti_all,
             [(h0, r0, t0), (h1, r1, t1)], part_v, out_v,
             sem_idx, [sem_a, sem_b])


def kernel(data, e_table, r_table):
    flat = data.astype(jnp.int32).reshape(TOTAL * 3)
    out = _distmult_sc(flat, e_table, r_table)
    return out.reshape(B, N)


# speedup vs baseline: 1.6220x; 1.6220x over previous
import functools

import jax
import jax.numpy as jnp
from jax import lax
from jax.experimental import pallas as pl
from jax.experimental.pallas import tpu as pltpu
from jax.experimental.pallas import tpu_sc as plsc

B = 1024
N = 64
D = 128
TOTAL = B * N
NC, NS, L = 2, 16, 16
NW = NC * NS
PER_W = TOTAL // NW
C = 128
NCH = PER_W // C


def _compute_chunk(h_rows, r_rows, t_rows, part_v, out_v, c, lane_iota):
    def group_body(g, _):
        def row_body(rr, _):
            i = g * L + rr
            acc = (h_rows[i, pl.ds(0, L)]
                   * r_rows[i, pl.ds(0, L)]
                   * t_rows[i, pl.ds(0, L)])
            for j in range(1, D // L):
                acc = acc + (h_rows[i, pl.ds(j * L, L)]
                             * r_rows[i, pl.ds(j * L, L)]
                             * t_rows[i, pl.ds(j * L, L)])
            plsc.store_scatter(part_v, [lane_iota * L + rr], acc)
            return 0

        lax.fori_loop(0, L, row_body, 0, unroll=4)

        tot = part_v[pl.ds(0, L)]
        for k in range(1, L):
            tot = tot + part_v[pl.ds(k * L, L)]
        out_v[pl.ds(c * C + g * L, L)] = tot
        return 0

    lax.fori_loop(0, C // L, group_body, 0)


def _sc_body(h_hbm, r_hbm, t_hbm, e_hbm, rel_hbm, out_hbm,
             hi_all, ri_all, ti_all, bufs, part_v, out_v,
             sem_idx, sems):
    wid = lax.axis_index("s") * NC + lax.axis_index("c")
    base = wid * PER_W
    lane_iota = lax.iota(jnp.int32, L)

    cps = [pltpu.async_copy(h_hbm.at[pl.ds(base, PER_W)], hi_all, sem_idx),
           pltpu.async_copy(r_hbm.at[pl.ds(base, PER_W)], ri_all, sem_idx),
           pltpu.async_copy(t_hbm.at[pl.ds(base, PER_W)], ti_all, sem_idx)]
    for cp in cps:
        cp.wait()

    def fire(c, b):
        sl = pl.ds(c * C, C)
        h_rows, r_rows, t_rows = bufs[b]
        return [
            pltpu.async_copy(e_hbm.at[hi_all.at[sl]], h_rows, sems[b]),
            pltpu.async_copy(rel_hbm.at[ri_all.at[sl]], r_rows, sems[b]),
            pltpu.async_copy(e_hbm.at[ti_all.at[sl]], t_rows, sems[b]),
        ]

    pending = {0: fire(0, 0)}
    for c in range(NCH):
        b = c % 2
        if c + 1 < NCH:
            pending[c + 1] = fire(c + 1, 1 - b)
        for cp in pending.pop(c):
            cp.wait()
        h_rows, r_rows, t_rows = bufs[b]
        _compute_chunk(h_rows, r_rows, t_rows, part_v, out_v, c, lane_iota)

    pltpu.sync_copy(out_v, out_hbm.at[pl.ds(base, PER_W)])


@functools.partial(
    pl.kernel,
    out_type=jax.ShapeDtypeStruct((TOTAL,), jnp.float32),
    mesh=plsc.VectorSubcoreMesh(core_axis_name="c", subcore_axis_name="s"),
    compiler_params=pltpu.CompilerParams(needs_layout_passes=False),
    scratch_types=[
        pltpu.VMEM((PER_W,), jnp.int32),
        pltpu.VMEM((PER_W,), jnp.int32),
        pltpu.VMEM((PER_W,), jnp.int32),
        pltpu.VMEM((C, D), jnp.float32),
        pltpu.VMEM((C, D), jnp.float32),
        pltpu.VMEM((C, D), jnp.float32),
        pltpu.VMEM((C, D), jnp.float32),
        pltpu.VMEM((C, D), jnp.float32),
        pltpu.VMEM((C, D), jnp.float32),
        pltpu.VMEM((L * L,), jnp.float32),
        pltpu.VMEM((PER_W,), jnp.float32),
        pltpu.SemaphoreType.DMA,
        pltpu.SemaphoreType.DMA,
        pltpu.SemaphoreType.DMA,
    ],
)
def _distmult_sc(h_hbm, r_hbm, t_hbm, e_hbm, rel_hbm, out_hbm,
                 hi_all, ri_all, ti_all,
                 h0, r0, t0, h1, r1, t1, part_v, out_v,
                 sem_idx, sem_a, sem_b):
    _sc_body(h_hbm, r_hbm, t_hbm, e_hbm, rel_hbm, out_hbm,
             hi_all, ri_all, ti_all,
             [(h0, r0, t0), (h1, r1, t1)], part_v, out_v,
             sem_idx, [sem_a, sem_b])


def kernel(data, e_table, r_table):
    flat = data.reshape(TOTAL, 3)
    h_idx = flat[:, 0].astype(jnp.int32)
    r_idx = flat[:, 1].astype(jnp.int32)
    t_idx = flat[:, 2].astype(jnp.int32)
    out = _distmult_sc(h_idx, r_idx, t_idx, e_table, r_table)
    return out.reshape(B, N)


# speedup vs baseline: 1.6261x; 1.0025x over previous
import functools

import jax
import jax.numpy as jnp
from jax import lax
from jax.experimental import pallas as pl
from jax.experimental.pallas import tpu as pltpu
from jax.experimental.pallas import tpu_sc as plsc

B = 1024
N = 64
D = 128
TOTAL = B * N
NC, NS, L = 2, 16, 16
NW = NC * NS
PER_W = TOTAL // NW
C = 64
NCH = PER_W // C
DEPTH = 4


def _compute_chunk(h_rows, r_rows, t_rows, part_v, out_v, c, lane_iota):
    def group_body(g, _):
        def row_body(rr, _):
            i = g * L + rr
            acc = (h_rows[i, pl.ds(0, L)]
                   * r_rows[i, pl.ds(0, L)]
                   * t_rows[i, pl.ds(0, L)])
            for j in range(1, D // L):
                acc = acc + (h_rows[i, pl.ds(j * L, L)]
                             * r_rows[i, pl.ds(j * L, L)]
                             * t_rows[i, pl.ds(j * L, L)])
            plsc.store_scatter(part_v, [lane_iota * L + rr], acc)
            return 0

        lax.fori_loop(0, L, row_body, 0)

        tot = part_v[pl.ds(0, L)]
        for k in range(1, L):
            tot = tot + part_v[pl.ds(k * L, L)]
        out_v[pl.ds(c * C + g * L, L)] = tot
        return 0

    lax.fori_loop(0, C // L, group_body, 0)


def _sc_body(h_hbm, r_hbm, t_hbm, e_hbm, rel_hbm, out_hbm,
             hi_all, ri_all, ti_all, bufs, part_v, out_v,
             sem_idx, sems):
    wid = lax.axis_index("s") * NC + lax.axis_index("c")
    base = wid * PER_W
    lane_iota = lax.iota(jnp.int32, L)

    cps = [pltpu.async_copy(h_hbm.at[pl.ds(base, PER_W)], hi_all, sem_idx),
           pltpu.async_copy(r_hbm.at[pl.ds(base, PER_W)], ri_all, sem_idx),
           pltpu.async_copy(t_hbm.at[pl.ds(base, PER_W)], ti_all, sem_idx)]
    for cp in cps:
        cp.wait()

    def fire(c):
        sl = pl.ds(c * C, C)
        h_rows, r_rows, t_rows = bufs[c % DEPTH]
        sem = sems[c % DEPTH]
        return [
            pltpu.async_copy(e_hbm.at[hi_all.at[sl]], h_rows, sem),
            pltpu.async_copy(rel_hbm.at[ri_all.at[sl]], r_rows, sem),
            pltpu.async_copy(e_hbm.at[ti_all.at[sl]], t_rows, sem),
        ]

    pending = {}
    for c in range(DEPTH - 1):
        pending[c] = fire(c)
    for c in range(NCH):
        if c + DEPTH - 1 < NCH:
            pending[c + DEPTH - 1] = fire(c + DEPTH - 1)
        for cp in pending.pop(c):
            cp.wait()
        h_rows, r_rows, t_rows = bufs[c % DEPTH]
        _compute_chunk(h_rows, r_rows, t_rows, part_v, out_v, c, lane_iota)

    pltpu.sync_copy(out_v, out_hbm.at[pl.ds(base, PER_W)])


@functools.partial(
    pl.kernel,
    out_type=jax.ShapeDtypeStruct((TOTAL,), jnp.float32),
    mesh=plsc.VectorSubcoreMesh(core_axis_name="c", subcore_axis_name="s"),
    compiler_params=pltpu.CompilerParams(needs_layout_passes=False),
    scratch_types=[
        pltpu.VMEM((PER_W,), jnp.int32),
        pltpu.VMEM((PER_W,), jnp.int32),
        pltpu.VMEM((PER_W,), jnp.int32),
    ] + [pltpu.VMEM((C, D), jnp.float32) for _ in range(3 * DEPTH)] + [
        pltpu.VMEM((L * L,), jnp.float32),
        pltpu.VMEM((PER_W,), jnp.float32),
    ] + [pltpu.SemaphoreType.DMA for _ in range(DEPTH + 1)],
)
def _distmult_sc(h_hbm, r_hbm, t_hbm, e_hbm, rel_hbm, out_hbm,
                 hi_all, ri_all, ti_all, *rest):
    rowbufs = rest[:3 * DEPTH]
    part_v = rest[3 * DEPTH]
    out_v = rest[3 * DEPTH + 1]
    sem_idx = rest[3 * DEPTH + 2]
    sems = rest[3 * DEPTH + 3:]
    bufs = [tuple(rowbufs[3 * d:3 * d + 3]) for d in range(DEPTH)]
    _sc_body(h_hbm, r_hbm, t_hbm, e_hbm, rel_hbm, out_hbm,
             hi_all, ri_all, ti_all, bufs, part_v, out_v,
             sem_idx, list(sems))


def kernel(data, e_table, r_table):
    flat = data.reshape(TOTAL, 3)
    h_idx = flat[:, 0].astype(jnp.int32)
    r_idx = flat[:, 1].astype(jnp.int32)
    t_idx = flat[:, 2].astype(jnp.int32)
    out = _distmult_sc(h_idx, r_idx, t_idx, e_table, r_table)
    return out.reshape(B, N)


# speedup vs baseline: 1.6723x; 1.0284x over previous
import functools

import jax
import jax.numpy as jnp
from jax import lax
from jax.experimental import pallas as pl
from jax.experimental.pallas import tpu as pltpu
from jax.experimental.pallas import tpu_sc as plsc

B = 1024
N = 64
D = 128
TOTAL = B * N
NC, NS, L = 2, 16, 16
NW = NC * NS
PER_W = TOTAL // NW
C = 128
NCH = PER_W // C
DEPTH = 2


def _compute_chunk(h_rows, r_rows, t_rows, part_v, out_v, c, lane_iota):
    def group_body(g, _):
        def row_body(rr, _):
            i = g * L + rr
            acc = (h_rows[i, pl.ds(0, L)]
                   * r_rows[i, pl.ds(0, L)]
                   * t_rows[i, pl.ds(0, L)])
            for j in range(1, D // L):
                acc = acc + (h_rows[i, pl.ds(j * L, L)]
                             * r_rows[i, pl.ds(j * L, L)]
                             * t_rows[i, pl.ds(j * L, L)])
            plsc.store_scatter(part_v, [lane_iota * L + rr], acc)
            return 0

        lax.fori_loop(0, L, row_body, 0)

        tot = part_v[pl.ds(0, L)]
        for k in range(1, L):
            tot = tot + part_v[pl.ds(k * L, L)]
        out_v[pl.ds(c * C + g * L, L)] = tot
        return 0

    lax.fori_loop(0, C // L, group_body, 0)


def _sc_body(h_hbm, r_hbm, t_hbm, e_hbm, rel_hbm, out_hbm,
             hi_all, ri_all, ti_all, bufs, part_v, out_v,
             sem_idx, sems):
    wid = lax.axis_index("s") * NC + lax.axis_index("c")
    base = wid * PER_W
    lane_iota = lax.iota(jnp.int32, L)

    cps = [pltpu.async_copy(h_hbm.at[pl.ds(base, PER_W)], hi_all, sem_idx),
           pltpu.async_copy(r_hbm.at[pl.ds(base, PER_W)], ri_all, sem_idx),
           pltpu.async_copy(t_hbm.at[pl.ds(base, PER_W)], ti_all, sem_idx)]
    for cp in cps:
        cp.wait()

    def fire(c):
        sl = pl.ds(c * C, C)
        h_rows, r_rows, t_rows = bufs[c % DEPTH]
        sem = sems[c % DEPTH]
        return [
            pltpu.async_copy(e_hbm.at[hi_all.at[sl]], h_rows, sem),
            pltpu.async_copy(rel_hbm.at[ri_all.at[sl]], r_rows, sem),
            pltpu.async_copy(e_hbm.at[ti_all.at[sl]], t_rows, sem),
        ]

    pending = {}
    for c in range(DEPTH - 1):
        pending[c] = fire(c)
    for c in range(NCH):
        if c + DEPTH - 1 < NCH:
            pending[c + DEPTH - 1] = fire(c + DEPTH - 1)
        for cp in pending.pop(c):
            cp.wait()
        h_rows, r_rows, t_rows = bufs[c % DEPTH]
        _compute_chunk(h_rows, r_rows, t_rows, part_v, out_v, c, lane_iota)

    pltpu.sync_copy(out_v, out_hbm.at[pl.ds(base, PER_W)])


@functools.partial(
    pl.kernel,
    out_type=jax.ShapeDtypeStruct((TOTAL,), jnp.float32),
    mesh=plsc.VectorSubcoreMesh(core_axis_name="c", subcore_axis_name="s"),
    compiler_params=pltpu.CompilerParams(needs_layout_passes=False),
    scratch_types=[
        pltpu.VMEM((PER_W,), jnp.int32),
        pltpu.VMEM((PER_W,), jnp.int32),
        pltpu.VMEM((PER_W,), jnp.int32),
    ] + [pltpu.VMEM((C, D), jnp.float32) for _ in range(3 * DEPTH)] + [
        pltpu.VMEM((L * L,), jnp.float32),
        pltpu.VMEM((PER_W,), jnp.float32),
    ] + [pltpu.SemaphoreType.DMA for _ in range(DEPTH + 1)],
)
def _distmult_sc(h_hbm, r_hbm, t_hbm, e_hbm, rel_hbm, out_hbm,
                 hi_all, ri_all, ti_all, *rest):
    rowbufs = rest[:3 * DEPTH]
    part_v = rest[3 * DEPTH]
    out_v = rest[3 * DEPTH + 1]
    sem_idx = rest[3 * DEPTH + 2]
    sems = rest[3 * DEPTH + 3:]
    bufs = [tuple(rowbufs[3 * d:3 * d + 3]) for d in range(DEPTH)]
    _sc_body(h_hbm, r_hbm, t_hbm, e_hbm, rel_hbm, out_hbm,
             hi_all, ri_all, ti_all, bufs, part_v, out_v,
             sem_idx, list(sems))


def kernel(data, e_table, r_table):
    flat = data.reshape(TOTAL, 3)
    h_idx = flat[:, 0].astype(jnp.int32)
    r_idx = flat[:, 1].astype(jnp.int32)
    t_idx = flat[:, 2].astype(jnp.int32)
    out = _distmult_sc(h_idx, r_idx, t_idx, e_table, r_table)
    return out.reshape(B, N)


# speedup vs baseline: 1.6997x; 1.0164x over previous
import functools

import jax
import jax.numpy as jnp
from jax import lax
from jax.experimental import pallas as pl
from jax.experimental.pallas import tpu as pltpu
from jax.experimental.pallas import tpu_sc as plsc

B = 1024
N = 64
D = 128
TOTAL = B * N
NC, NS, L = 2, 16, 16
NW = NC * NS
PER_W = TOTAL // NW
C = 128
NCH = PER_W // C
DEPTH = 2


def _compute_chunk(h_rows, r_rows, t_rows, part_v, out_v, c, lane_iota):
    def group_body(g, _):
        def row_body(rr, _):
            i = g * L + rr
            acc = (h_rows[i, pl.ds(0, L)]
                   * r_rows[i, pl.ds(0, L)]
                   * t_rows[i, pl.ds(0, L)])
            for j in range(1, D // L):
                acc = acc + (h_rows[i, pl.ds(j * L, L)]
                             * r_rows[i, pl.ds(j * L, L)]
                             * t_rows[i, pl.ds(j * L, L)])
            plsc.store_scatter(part_v, [lane_iota * L + rr], acc)
            return 0

        lax.fori_loop(0, L, row_body, 0)

        tot = part_v[pl.ds(0, L)]
        for k in range(1, L):
            tot = tot + part_v[pl.ds(k * L, L)]
        out_v[pl.ds(c * C + g * L, L)] = tot
        return 0

    lax.fori_loop(0, C // L, group_body, 0)


def _sc_body(h_hbm, r_hbm, t_hbm, e_hbm, rel_hbm, out_hbm,
             hi_all, ri_all, ti_all, bufs, part_v, out_v,
             sem_idx, sems):
    wid = lax.axis_index("s") * NC + lax.axis_index("c")
    base = wid * PER_W
    lane_iota = lax.iota(jnp.int32, L)

    cps = [pltpu.async_copy(h_hbm.at[pl.ds(base, C)], hi_all.at[pl.ds(0, C)],
                            sem_idx),
           pltpu.async_copy(r_hbm.at[pl.ds(base, C)], ri_all.at[pl.ds(0, C)],
                            sem_idx),
           pltpu.async_copy(t_hbm.at[pl.ds(base, C)], ti_all.at[pl.ds(0, C)],
                            sem_idx)]
    for cp in cps:
        cp.wait()

    def fire(c):
        sl = pl.ds(c * C, C)
        h_rows, r_rows, t_rows = bufs[c % DEPTH]
        sem = sems[c % DEPTH]
        return [
            pltpu.async_copy(e_hbm.at[hi_all.at[sl]], h_rows, sem),
            pltpu.async_copy(rel_hbm.at[ri_all.at[sl]], r_rows, sem),
            pltpu.async_copy(e_hbm.at[ti_all.at[sl]], t_rows, sem),
        ]

    pending = {0: fire(0)}
    rest = PER_W - C
    cps = [pltpu.async_copy(h_hbm.at[pl.ds(base + C, rest)],
                            hi_all.at[pl.ds(C, rest)], sem_idx),
           pltpu.async_copy(r_hbm.at[pl.ds(base + C, rest)],
                            ri_all.at[pl.ds(C, rest)], sem_idx),
           pltpu.async_copy(t_hbm.at[pl.ds(base + C, rest)],
                            ti_all.at[pl.ds(C, rest)], sem_idx)]
    for cp in cps:
        cp.wait()
    for c in range(1, DEPTH - 1):
        pending[c] = fire(c)
    for c in range(NCH):
        if c + DEPTH - 1 < NCH:
            pending[c + DEPTH - 1] = fire(c + DEPTH - 1)
        for cp in pending.pop(c):
            cp.wait()
        h_rows, r_rows, t_rows = bufs[c % DEPTH]
        _compute_chunk(h_rows, r_rows, t_rows, part_v, out_v, c, lane_iota)

    pltpu.sync_copy(out_v, out_hbm.at[pl.ds(base, PER_W)])


@functools.partial(
    pl.kernel,
    out_type=jax.ShapeDtypeStruct((TOTAL,), jnp.float32),
    mesh=plsc.VectorSubcoreMesh(core_axis_name="c", subcore_axis_name="s"),
    compiler_params=pltpu.CompilerParams(needs_layout_passes=False),
    scratch_types=[
        pltpu.VMEM((PER_W,), jnp.int32),
        pltpu.VMEM((PER_W,), jnp.int32),
        pltpu.VMEM((PER_W,), jnp.int32),
    ] + [pltpu.VMEM((C, D), jnp.float32) for _ in range(3 * DEPTH)] + [
        pltpu.VMEM((L * L,), jnp.float32),
        pltpu.VMEM((PER_W,), jnp.float32),
    ] + [pltpu.SemaphoreType.DMA for _ in range(DEPTH + 1)],
)
def _distmult_sc(h_hbm, r_hbm, t_hbm, e_hbm, rel_hbm, out_hbm,
                 hi_all, ri_all, ti_all, *rest):
    rowbufs = rest[:3 * DEPTH]
    part_v = rest[3 * DEPTH]
    out_v = rest[3 * DEPTH + 1]
    sem_idx = rest[3 * DEPTH + 2]
    sems = rest[3 * DEPTH + 3:]
    bufs = [tuple(rowbufs[3 * d:3 * d + 3]) for d in range(DEPTH)]
    _sc_body(h_hbm, r_hbm, t_hbm, e_hbm, rel_hbm, out_hbm,
             hi_all, ri_all, ti_all, bufs, part_v, out_v,
             sem_idx, list(sems))


def kernel(data, e_table, r_table):
    flat = data.reshape(TOTAL, 3)
    h_idx = flat[:, 0].astype(jnp.int32)
    r_idx = flat[:, 1].astype(jnp.int32)
    t_idx = flat[:, 2].astype(jnp.int32)
    out = _distmult_sc(h_idx, r_idx, t_idx, e_table, r_table)
    return out.reshape(B, N)
